# baseline (speedup 1.0000x reference)
import functools

import jax
import jax.numpy as jnp
from jax import lax
from jax.experimental import pallas as pl
from jax.experimental.pallas import tpu as pltpu
from jax.experimental.pallas import tpu_sc as plsc

N = 10000
E = 320000
D = 128

NC = 2
NS = 16
NW = NC * NS

EPW = E // NW
CH = 96
NCHUNK = EPW // CH
TAIL = EPW - NCHUNK * CH
R = 3
NSTEADY = 33

RPT = 624
ZB = 24
REM = N - NS * RPT


def _sc_body(x, src, dst, out,
             src_v, dst_v, rows_v, tsrc_v, tdst_v, trows_v, zbuf,
             acc, sem_i, sem_g, sem_s, sem_t, sem_z):
    cid = lax.axis_index("c")
    sid = lax.axis_index("s")
    wid = cid * NS + sid
    ebase = wid * EPW

    def issue_idx(c, q):
        pltpu.async_copy(src.at[pl.ds(ebase + c * CH, CH)], src_v[q], sem_i[q])
        pltpu.async_copy(dst.at[pl.ds(ebase + c * CH, CH)], dst_v[q], sem_i[q])

    def wait_idx(c, q):
        pltpu.make_async_copy(src.at[pl.ds(ebase + c * CH, CH)], src_v[q],
                              sem_i[q]).wait()
        pltpu.make_async_copy(dst.at[pl.ds(ebase + c * CH, CH)], dst_v[q],
                              sem_i[q]).wait()

    def issue_gather(q):
        pltpu.async_copy(x.at[src_v[q]], rows_v[q], sem_g[q])

    def wait_gather(q):
        pltpu.make_async_copy(x.at[src_v[q]], rows_v[q], sem_g[q]).wait()

    def issue_scatter(q):
        pltpu.async_copy(rows_v[q], acc.at[dst_v[q]], sem_s[q], add=True)

    def wait_scatter(q):
        pltpu.make_async_copy(rows_v[q], acc.at[dst_v[q]], sem_s[q]).wait()

    issue_idx(0, 0)
    issue_idx(1, 1)
    issue_idx(2, 2)
    tbase = ebase + NCHUNK * CH
    pltpu.async_copy(src.at[pl.ds(tbase, TAIL)], tsrc_v, sem_t)
    pltpu.async_copy(dst.at[pl.ds(tbase, TAIL)], tdst_v, sem_t)

    def _zero(r, _):
        for cb in range(D // 16):
            zbuf[r, pl.ds(cb * 16, 16)] = jnp.zeros((16,), jnp.float32)
        return 0
    lax.fori_loop(0, ZB, _zero, 0)

    for j in range(RPT // ZB):
        pltpu.async_copy(zbuf, acc.at[pl.ds(sid * RPT + j * ZB, ZB)], sem_z)

    @pl.when(sid == NS - 1)
    def _zero_rem():
        pltpu.async_copy(zbuf.at[pl.ds(0, REM)], acc.at[pl.ds(NS * RPT, REM)],
                         sem_z)

    wait_idx(0, 0)
    issue_gather(0)
    wait_idx(1, 1)
    issue_gather(1)

    for j in range(RPT // ZB):
        pltpu.make_async_copy(zbuf, acc.at[pl.ds(sid * RPT + j * ZB, ZB)],
                              sem_z).wait()

    @pl.when(sid == NS - 1)
    def _zero_rem_wait():
        pltpu.make_async_copy(zbuf.at[pl.ds(0, REM)],
                              acc.at[pl.ds(NS * RPT, REM)], sem_z).wait()

    plsc.subcore_barrier()

    def _steady(g, _):
        for k in range(R):
            c = g * R + k
            q = k % R
            wait_idx(c + 2, (k + 2) % R)
            issue_gather((k + 2) % R)
            wait_gather(q)
            issue_scatter(q)
            wait_scatter(q)
            issue_idx(c + 3, q)
        return 0
    lax.fori_loop(0, NSTEADY, _steady, 0)

    for c in range(NSTEADY * R, NCHUNK):
        q = c % R
        if c + 2 < NCHUNK:
            wait_idx(c + 2, (c + 2) % R)
            issue_gather((c + 2) % R)
        wait_gather(q)
        issue_scatter(q)
        wait_scatter(q)
        if c + 3 < NCHUNK:
            issue_idx(c + 3, (c + 3) % R)

    pltpu.make_async_copy(src.at[pl.ds(tbase, TAIL)], tsrc_v, sem_t).wait()
    pltpu.make_async_copy(dst.at[pl.ds(tbase, TAIL)], tdst_v, sem_t).wait()
    pltpu.async_copy(x.at[tsrc_v], trows_v, sem_t).wait()
    pltpu.sync_copy(trows_v, acc.at[tdst_v], add=True)

    plsc.subcore_barrier()

    r0 = sid * RPT
    pltpu.async_copy(acc.at[pl.ds(r0, RPT)],
                     out.at[pl.ds(cid * N + r0, RPT)], sem_t)
    pltpu.make_async_copy(acc.at[pl.ds(r0, RPT)],
                          out.at[pl.ds(cid * N + r0, RPT)], sem_t).wait()

    @pl.when(sid == NS - 1)
    def _write_rem():
        pltpu.async_copy(acc.at[pl.ds(NS * RPT, REM)],
                         out.at[pl.ds(cid * N + NS * RPT, REM)], sem_t)
        pltpu.make_async_copy(acc.at[pl.ds(NS * RPT, REM)],
                              out.at[pl.ds(cid * N + NS * RPT, REM)],
                              sem_t).wait()


_sc_scatter = functools.partial(
    pl.kernel,
    out_type=jax.ShapeDtypeStruct((NC * N, D), jnp.float32),
    mesh=plsc.VectorSubcoreMesh(core_axis_name="c", subcore_axis_name="s"),
    scratch_types=[
        [pltpu.VMEM((CH,), jnp.int32) for _ in range(R)],
        [pltpu.VMEM((CH,), jnp.int32) for _ in range(R)],
        [pltpu.VMEM((CH, D), jnp.float32) for _ in range(R)],
        pltpu.VMEM((TAIL,), jnp.int32),
        pltpu.VMEM((TAIL,), jnp.int32),
        pltpu.VMEM((TAIL, D), jnp.float32),
        pltpu.VMEM((ZB, D), jnp.float32),
        pltpu.VMEM_SHARED((N, D), jnp.float32),
        [pltpu.SemaphoreType.DMA for _ in range(R)],
        [pltpu.SemaphoreType.DMA for _ in range(R)],
        [pltpu.SemaphoreType.DMA for _ in range(R)],
        pltpu.SemaphoreType.DMA,
        pltpu.SemaphoreType.DMA,
    ],
)(_sc_body)


def _comb_body(p_ref, w_ref, b_ref, o_ref):
    o_ref[...] = jnp.dot(p_ref[0] + p_ref[1], w_ref[...],
                         preferred_element_type=jnp.float32,
                         precision=lax.Precision.HIGHEST) + b_ref[...]


def _combine_matmul(partial, w, bias):
    BM = 2000
    return pl.pallas_call(
        _comb_body,
        grid=(N // BM,),
        in_specs=[
            pl.BlockSpec((2, BM, D), lambda i: (0, i, 0)),
            pl.BlockSpec((D, D), lambda i: (0, 0)),
            pl.BlockSpec((1, D), lambda i: (0, 0)),
        ],
        out_specs=pl.BlockSpec((BM, D), lambda i: (i, 0)),
        out_shape=jax.ShapeDtypeStruct((N, D), jnp.float32),
    )(partial, w, bias)


def kernel(x, edge_index, weight, bias):
    src = edge_index[0]
    dst = edge_index[1]
    partial = _sc_scatter(x, src, dst)
    return _combine_matmul(partial.reshape(NC, N, D), weight,
                           bias.reshape(1, D))


# speedup vs baseline: 1.1488x; 1.1488x over previous
import functools

import jax
import jax.numpy as jnp
from jax import lax
from jax.experimental import pallas as pl
from jax.experimental.pallas import tpu as pltpu
from jax.experimental.pallas import tpu_sc as plsc

N = 10000
E = 320000
D = 128

NC = 2
NS = 16
NW = NC * NS

EPW = E // NW
CH = 128
NCHUNK = EPW // CH
TAIL = EPW - NCHUNK * CH
NBUF = 2
NIB = 3
UNROLL = 6
NSTEADY = (NCHUNK - UNROLL) // UNROLL

RPT = 624
ZB = 48
REM = N - NS * RPT


def _sc_body(x, src, dst, out,
             src_v, dst_v, rows_v, tsrc_v, tdst_v, trows_v, zbuf,
             acc, sem_i, sem_g, sem_s, sem_t, sem_z):
    cid = lax.axis_index("c")
    sid = lax.axis_index("s")
    wid = cid * NS + sid
    ebase = wid * EPW

    def issue_idx(c, b):
        pltpu.async_copy(src.at[pl.ds(ebase + c * CH, CH)], src_v[b], sem_i[b])
        pltpu.async_copy(dst.at[pl.ds(ebase + c * CH, CH)], dst_v[b], sem_i[b])

    def wait_idx(c, b):
        pltpu.make_async_copy(src.at[pl.ds(ebase + c * CH, CH)], src_v[b],
                              sem_i[b]).wait()
        pltpu.make_async_copy(dst.at[pl.ds(ebase + c * CH, CH)], dst_v[b],
                              sem_i[b]).wait()

    def issue_gather(b, q):
        pltpu.async_copy(x.at[src_v[q]], rows_v[b], sem_g[b])

    def wait_gather(b, q):
        pltpu.make_async_copy(x.at[src_v[q]], rows_v[b], sem_g[b]).wait()

    def issue_scatter(b, q):
        pltpu.async_copy(rows_v[b], acc.at[dst_v[q]], sem_s[b], add=True)

    def wait_scatter(b, q):
        pltpu.make_async_copy(rows_v[b], acc.at[dst_v[q]], sem_s[b]).wait()

    issue_idx(0, 0)
    issue_idx(1, 1)
    issue_idx(2, 2)
    tbase = ebase + NCHUNK * CH
    pltpu.async_copy(src.at[pl.ds(tbase, TAIL)], tsrc_v, sem_t)
    pltpu.async_copy(dst.at[pl.ds(tbase, TAIL)], tdst_v, sem_t)

    def _zero(r, _):
        for cb in range(D // 16):
            zbuf[r, pl.ds(cb * 16, 16)] = jnp.zeros((16,), jnp.float32)
        return 0
    lax.fori_loop(0, ZB, _zero, 0)

    for j in range(RPT // ZB):
        pltpu.async_copy(zbuf, acc.at[pl.ds(sid * RPT + j * ZB, ZB)], sem_z)

    @pl.when(sid == NS - 1)
    def _zero_rem():
        pltpu.async_copy(zbuf.at[pl.ds(0, REM)], acc.at[pl.ds(NS * RPT, REM)],
                         sem_z)

    wait_idx(0, 0)
    issue_gather(0, 0)

    for j in range(RPT // ZB):
        pltpu.make_async_copy(zbuf, acc.at[pl.ds(sid * RPT + j * ZB, ZB)],
                              sem_z).wait()

    @pl.when(sid == NS - 1)
    def _zero_rem_wait():
        pltpu.make_async_copy(zbuf.at[pl.ds(0, REM)],
                              acc.at[pl.ds(NS * RPT, REM)], sem_z).wait()

    plsc.subcore_barrier()

    def _steady(g, _):
        for k in range(UNROLL):
            c = g * UNROLL + k
            b = k % NBUF
            wait_idx(c + 1, (k + 1) % NIB)
            issue_gather((k + 1) % NBUF, (k + 1) % NIB)
            wait_gather(b, k % NIB)
            issue_scatter(b, k % NIB)
            wait_scatter(b, k % NIB)
            issue_idx(c + 3, k % NIB)
        return 0
    lax.fori_loop(0, NSTEADY, _steady, 0)

    for k in range(UNROLL):
        c = NSTEADY * UNROLL + k
        b = c % NBUF
        if c + 1 < NCHUNK:
            wait_idx(c + 1, (c + 1) % NIB)
            issue_gather((c + 1) % NBUF, (c + 1) % NIB)
        wait_gather(b, c % NIB)
        issue_scatter(b, c % NIB)
        wait_scatter(b, c % NIB)
        if c + 3 < NCHUNK:
            issue_idx(c + 3, (c + 3) % NIB)

    pltpu.make_async_copy(src.at[pl.ds(tbase, TAIL)], tsrc_v, sem_t).wait()
    pltpu.make_async_copy(dst.at[pl.ds(tbase, TAIL)], tdst_v, sem_t).wait()
    pltpu.async_copy(x.at[tsrc_v], trows_v, sem_t).wait()
    pltpu.sync_copy(trows_v, acc.at[tdst_v], add=True)

    plsc.subcore_barrier()

    r0 = sid * RPT
    pltpu.async_copy(acc.at[pl.ds(r0, RPT)],
                     out.at[pl.ds(cid * N + r0, RPT)], sem_t)
    pltpu.make_async_copy(acc.at[pl.ds(r0, RPT)],
                          out.at[pl.ds(cid * N + r0, RPT)], sem_t).wait()

    @pl.when(sid == NS - 1)
    def _write_rem():
        pltpu.async_copy(acc.at[pl.ds(NS * RPT, REM)],
                         out.at[pl.ds(cid * N + NS * RPT, REM)], sem_t)
        pltpu.make_async_copy(acc.at[pl.ds(NS * RPT, REM)],
                              out.at[pl.ds(cid * N + NS * RPT, REM)],
                              sem_t).wait()


_sc_scatter = functools.partial(
    pl.kernel,
    out_type=jax.ShapeDtypeStruct((NC * N, D), jnp.float32),
    mesh=plsc.VectorSubcoreMesh(core_axis_name="c", subcore_axis_name="s"),
    scratch_types=[
        [pltpu.VMEM((CH,), jnp.int32) for _ in range(NIB)],
        [pltpu.VMEM((CH,), jnp.int32) for _ in range(NIB)],
        [pltpu.VMEM((CH, D), jnp.float32) for _ in range(NBUF)],
        pltpu.VMEM((TAIL,), jnp.int32),
        pltpu.VMEM((TAIL,), jnp.int32),
        pltpu.VMEM((TAIL, D), jnp.float32),
        pltpu.VMEM((ZB, D), jnp.float32),
        pltpu.VMEM_SHARED((N, D), jnp.float32),
        [pltpu.SemaphoreType.DMA for _ in range(NIB)],
        [pltpu.SemaphoreType.DMA for _ in range(NBUF)],
        [pltpu.SemaphoreType.DMA for _ in range(NBUF)],
        pltpu.SemaphoreType.DMA,
        pltpu.SemaphoreType.DMA,
    ],
)(_sc_body)


def _comb_body(p_ref, w_ref, b_ref, o_ref):
    o_ref[...] = jnp.dot(p_ref[0] + p_ref[1], w_ref[...],
                         preferred_element_type=jnp.float32,
                         precision=lax.Precision.HIGHEST) + b_ref[...]


def _combine_matmul(partial, w, bias):
    BM = 2000
    return pl.pallas_call(
        _comb_body,
        grid=(N // BM,),
        in_specs=[
            pl.BlockSpec((2, BM, D), lambda i: (0, i, 0)),
            pl.BlockSpec((D, D), lambda i: (0, 0)),
            pl.BlockSpec((1, D), lambda i: (0, 0)),
        ],
        out_specs=pl.BlockSpec((BM, D), lambda i: (i, 0)),
        out_shape=jax.ShapeDtypeStruct((N, D), jnp.float32),
    )(partial, w, bias)


def kernel(x, edge_index, weight, bias):
    src = edge_index[0]
    dst = edge_index[1]
    partial = _sc_scatter(x, src, dst)
    return _combine_matmul(partial.reshape(NC, N, D), weight,
                           bias.reshape(1, D))


# speedup vs baseline: 1.1493x; 1.0004x over previous
import functools

import jax
import jax.numpy as jnp
from jax import lax
from jax.experimental import pallas as pl
from jax.experimental.pallas import tpu as pltpu
from jax.experimental.pallas import tpu_sc as plsc

N = 10000
E = 320000
D = 128

NC = 2
NS = 16
NW = NC * NS

EPW = E // NW
CH = 128
NCHUNK = EPW // CH
TAIL = EPW - NCHUNK * CH
NBUF = 2
NIB = 4
UNROLL = 4
NSTEADY = 18

RPT = 624
ZB = 48
REM = N - NS * RPT


def _sc_body(x, src, dst, out,
             src_v, dst_v, rows_v, tsrc_v, tdst_v, trows_v, zbuf,
             acc, sem_i, sem_g, sem_s, sem_t, sem_z):
    cid = lax.axis_index("c")
    sid = lax.axis_index("s")
    wid = cid * NS + sid
    ebase = wid * EPW

    def issue_idx(c, b):
        pltpu.async_copy(src.at[pl.ds(ebase + c * CH, CH)], src_v[b], sem_i[b])
        pltpu.async_copy(dst.at[pl.ds(ebase + c * CH, CH)], dst_v[b], sem_i[b])

    def wait_idx(c, b):
        pltpu.make_async_copy(src.at[pl.ds(ebase + c * CH, CH)], src_v[b],
                              sem_i[b]).wait()
        pltpu.make_async_copy(dst.at[pl.ds(ebase + c * CH, CH)], dst_v[b],
                              sem_i[b]).wait()

    def issue_gather(b, q):
        pltpu.async_copy(x.at[src_v[q]], rows_v[b], sem_g[b])

    def wait_gather(b, q):
        pltpu.make_async_copy(x.at[src_v[q]], rows_v[b], sem_g[b]).wait()

    def issue_scatter(b, q):
        pltpu.async_copy(rows_v[b], acc.at[dst_v[q]], sem_s[b], add=True)

    def wait_scatter(b, q):
        pltpu.make_async_copy(rows_v[b], acc.at[dst_v[q]], sem_s[b]).wait()

    issue_idx(0, 0)
    issue_idx(1, 1)
    issue_idx(2, 2)
    issue_idx(3, 3)
    tbase = ebase + NCHUNK * CH
    pltpu.async_copy(src.at[pl.ds(tbase, TAIL)], tsrc_v, sem_t)
    pltpu.async_copy(dst.at[pl.ds(tbase, TAIL)], tdst_v, sem_t)

    def _zero(r, _):
        for cb in range(D // 16):
            zbuf[r, pl.ds(cb * 16, 16)] = jnp.zeros((16,), jnp.float32)
        return 0
    lax.fori_loop(0, ZB, _zero, 0)

    for j in range(RPT // ZB):
        pltpu.async_copy(zbuf, acc.at[pl.ds(sid * RPT + j * ZB, ZB)], sem_z)

    @pl.when(sid == NS - 1)
    def _zero_rem():
        pltpu.async_copy(zbuf.at[pl.ds(0, REM)], acc.at[pl.ds(NS * RPT, REM)],
                         sem_z)

    wait_idx(0, 0)
    issue_gather(0, 0)

    for j in range(RPT // ZB):
        pltpu.make_async_copy(zbuf, acc.at[pl.ds(sid * RPT + j * ZB, ZB)],
                              sem_z).wait()

    @pl.when(sid == NS - 1)
    def _zero_rem_wait():
        pltpu.make_async_copy(zbuf.at[pl.ds(0, REM)],
                              acc.at[pl.ds(NS * RPT, REM)], sem_z).wait()

    plsc.subcore_barrier()

    wait_idx(1, 1)
    issue_gather(1, 1)
    wait_gather(0, 0)
    issue_scatter(0, 0)

    def _steady(g, _):
        for k in range(UNROLL):
            c = g * UNROLL + k + 1
            b = (k + 1) % NBUF
            wait_idx(c + 1, (k + 2) % NIB)
            wait_scatter(1 - b, k % NIB)
            issue_gather(1 - b, (k + 2) % NIB)
            wait_gather(b, (k + 1) % NIB)
            issue_scatter(b, (k + 1) % NIB)
            issue_idx(c + 3, k % NIB)
        return 0
    lax.fori_loop(0, NSTEADY, _steady, 0)

    for c in range(NSTEADY * UNROLL + 1, NCHUNK):
        b = c % NBUF
        if c + 1 < NCHUNK:
            wait_idx(c + 1, (c + 1) % NIB)
        wait_scatter(1 - b, (c - 1) % NIB)
        if c + 1 < NCHUNK:
            issue_gather((c + 1) % NBUF, (c + 1) % NIB)
        wait_gather(b, c % NIB)
        issue_scatter(b, c % NIB)
        if c + 3 < NCHUNK:
            issue_idx(c + 3, (c + 3) % NIB)
    wait_scatter((NCHUNK - 1) % NBUF, (NCHUNK - 1) % NIB)

    pltpu.make_async_copy(src.at[pl.ds(tbase, TAIL)], tsrc_v, sem_t).wait()
    pltpu.make_async_copy(dst.at[pl.ds(tbase, TAIL)], tdst_v, sem_t).wait()
    pltpu.async_copy(x.at[tsrc_v], trows_v, sem_t).wait()
    pltpu.sync_copy(trows_v, acc.at[tdst_v], add=True)

    plsc.subcore_barrier()

    r0 = sid * RPT
    pltpu.async_copy(acc.at[pl.ds(r0, RPT)],
                     out.at[pl.ds(cid * N + r0, RPT)], sem_t)
    pltpu.make_async_copy(acc.at[pl.ds(r0, RPT)],
                          out.at[pl.ds(cid * N + r0, RPT)], sem_t).wait()

    @pl.when(sid == NS - 1)
    def _write_rem():
        pltpu.async_copy(acc.at[pl.ds(NS * RPT, REM)],
                         out.at[pl.ds(cid * N + NS * RPT, REM)], sem_t)
        pltpu.make_async_copy(acc.at[pl.ds(NS * RPT, REM)],
                              out.at[pl.ds(cid * N + NS * RPT, REM)],
                              sem_t).wait()


_sc_scatter = functools.partial(
    pl.kernel,
    out_type=jax.ShapeDtypeStruct((NC * N, D), jnp.float32),
    mesh=plsc.VectorSubcoreMesh(core_axis_name="c", subcore_axis_name="s"),
    scratch_types=[
        [pltpu.VMEM((CH,), jnp.int32) for _ in range(NIB)],
        [pltpu.VMEM((CH,), jnp.int32) for _ in range(NIB)],
        [pltpu.VMEM((CH, D), jnp.float32) for _ in range(NBUF)],
        pltpu.VMEM((TAIL,), jnp.int32),
        pltpu.VMEM((TAIL,), jnp.int32),
        pltpu.VMEM((TAIL, D), jnp.float32),
        pltpu.VMEM((ZB, D), jnp.float32),
        pltpu.VMEM_SHARED((N, D), jnp.float32),
        [pltpu.SemaphoreType.DMA for _ in range(NIB)],
        [pltpu.SemaphoreType.DMA for _ in range(NBUF)],
        [pltpu.SemaphoreType.DMA for _ in range(NBUF)],
        pltpu.SemaphoreType.DMA,
        pltpu.SemaphoreType.DMA,
    ],
)(_sc_body)


def _comb_body(p_ref, w_ref, b_ref, o_ref):
    o_ref[...] = jnp.dot(p_ref[0] + p_ref[1], w_ref[...],
                         preferred_element_type=jnp.float32,
                         precision=lax.Precision.HIGHEST) + b_ref[...]


def _combine_matmul(partial, w, bias):
    BM = 2000
    return pl.pallas_call(
        _comb_body,
        grid=(N // BM,),
        in_specs=[
            pl.BlockSpec((2, BM, D), lambda i: (0, i, 0)),
            pl.BlockSpec((D, D), lambda i: (0, 0)),
            pl.BlockSpec((1, D), lambda i: (0, 0)),
        ],
        out_specs=pl.BlockSpec((BM, D), lambda i: (i, 0)),
        out_shape=jax.ShapeDtypeStruct((N, D), jnp.float32),
    )(partial, w, bias)


def kernel(x, edge_index, weight, bias):
    src = edge_index[0]
    dst = edge_index[1]
    partial = _sc_scatter(x, src, dst)
    return _combine_matmul(partial.reshape(NC, N, D), weight,
                           bias.reshape(1, D))
